# Initial kernel scaffold; baseline (speedup 1.0000x reference)
import functools

import jax
import jax.numpy as jnp
from jax import lax
from jax.experimental import pallas as pl
from jax.experimental.pallas import tpu as pltpu
from jax.experimental.pallas import tpu_sc as plsc

N = 10000
D = 128
E = 320000

NUM_CORES = 2
NUM_SUBCORES = 16
NUM_WORKERS = NUM_CORES * NUM_SUBCORES

CHUNK = 128
CHUNKS_PER_W = -(-E // (NUM_WORKERS * CHUNK))
E_PAD = NUM_WORKERS * CHUNKS_PER_W * CHUNK

AGG_ROWS = 10240
ZROWS = AGG_ROWS // NUM_SUBCORES
ROWS_PER_TILE = N // NUM_SUBCORES
DUMMY_DST = N + 8


def _mm_body(h_ref, norm_ref, w_ref, b_ref, out_ref):
    acc = jnp.dot(h_ref[...], w_ref[...], preferred_element_type=jnp.float32)
    out_ref[...] = (acc + b_ref[...]) * norm_ref[...]


def _linear(h, norm, W, b):
    blk = 1000
    grid = N // blk
    return pl.pallas_call(
        _mm_body,
        grid=(grid,),
        in_specs=[
            pl.BlockSpec((blk, D), lambda i: (i, 0)),
            pl.BlockSpec((blk, 1), lambda i: (i, 0)),
            pl.BlockSpec((D, D), lambda i: (0, 0)),
            pl.BlockSpec((1, D), lambda i: (0, 0)),
        ],
        out_specs=pl.BlockSpec((blk, D), lambda i: (i, 0)),
        out_shape=jax.ShapeDtypeStruct((N, D), jnp.float32),
    )(h, norm, W, b.reshape(1, D))


def _sc_agg_body(hh_hbm, src_hbm, dst_hbm, zeros_hbm, out_hbm,
                 sidx_v, didx_v, rows_v, agg_s, sem):
    c = lax.axis_index("c")
    s = lax.axis_index("s")

    def zero_body(k, carry):
        pltpu.sync_copy(zeros_hbm, agg_s.at[pl.ds(s * ZROWS + k * CHUNK, CHUNK)])
        return carry
    lax.fori_loop(0, ZROWS // CHUNK, zero_body, 0)
    plsc.subcore_barrier()

    w = c * NUM_SUBCORES + s
    base = w * CHUNKS_PER_W * CHUNK

    def edge_body(j, carry):
        off = base + j * CHUNK
        pltpu.sync_copy(src_hbm.at[pl.ds(off, CHUNK)], sidx_v)
        pltpu.sync_copy(dst_hbm.at[pl.ds(off, CHUNK)], didx_v)
        pltpu.async_copy(hh_hbm.at[sidx_v], rows_v, sem).wait()
        pltpu.sync_copy(rows_v, agg_s.at[didx_v], add=True)
        return carry
    lax.fori_loop(0, CHUNKS_PER_W, edge_body, 0)
    plsc.subcore_barrier()

    r0 = s * ROWS_PER_TILE
    pltpu.sync_copy(agg_s.at[pl.ds(r0, ROWS_PER_TILE)],
                    out_hbm.at[c].at[pl.ds(r0, ROWS_PER_TILE)])


_sc_agg = functools.partial(
    pl.kernel,
    out_type=jax.ShapeDtypeStruct((NUM_CORES, N, D), jnp.float32),
    mesh=plsc.VectorSubcoreMesh(core_axis_name="c", subcore_axis_name="s"),
    scratch_types=[
        pltpu.VMEM((CHUNK,), jnp.int32),
        pltpu.VMEM((CHUNK,), jnp.int32),
        pltpu.VMEM((CHUNK, D), jnp.float32),
        pltpu.VMEM_SHARED((AGG_ROWS, D), jnp.float32),
        pltpu.SemaphoreType.DMA,
    ],
)(_sc_agg_body)


def _comb_body(p_ref, norm_ref, o_ref):
    o_ref[...] = (p_ref[0] + p_ref[1]) * norm_ref[...]


def _combine(partials, norm):
    blk = 1000
    grid = N // blk
    return pl.pallas_call(
        _comb_body,
        grid=(grid,),
        in_specs=[
            pl.BlockSpec((NUM_CORES, blk, D), lambda i: (0, i, 0)),
            pl.BlockSpec((blk, 1), lambda i: (i, 0)),
        ],
        out_specs=pl.BlockSpec((blk, D), lambda i: (i, 0)),
        out_shape=jax.ShapeDtypeStruct((N, D), jnp.float32),
    )(partials, norm)


def kernel(h, norm, W, b, edge_index):
    hh = _linear(h, norm, W, b)

    src = edge_index[0].astype(jnp.int32)
    dst = edge_index[1].astype(jnp.int32)
    pad = E_PAD - E
    src = jnp.concatenate([src, jnp.zeros((pad,), jnp.int32)])
    dst = jnp.concatenate([dst, jnp.full((pad,), DUMMY_DST, jnp.int32)])
    zeros = jnp.zeros((CHUNK, D), jnp.float32)

    partials = _sc_agg(hh, src, dst, zeros)
    return _combine(partials, norm)


# speedup vs baseline: 4.1043x; 4.1043x over previous
import functools

import jax
import jax.numpy as jnp
from jax import lax
from jax.experimental import pallas as pl
from jax.experimental.pallas import tpu as pltpu
from jax.experimental.pallas import tpu_sc as plsc

N = 10000
D = 128
E = 320000

NUM_CORES = 2
NUM_SUBCORES = 16
NUM_WORKERS = NUM_CORES * NUM_SUBCORES

CHUNK = 128
CHUNKS_PER_W = -(-E // (NUM_WORKERS * CHUNK))
E_PAD = NUM_WORKERS * CHUNKS_PER_W * CHUNK

AGG_ROWS = 10240
ZROWS = AGG_ROWS // NUM_SUBCORES
DUMMY_DST = N + 8


def _mm_body(h_ref, norm_ref, w_ref, b_ref, out_ref):
    acc = jnp.dot(h_ref[...], w_ref[...], preferred_element_type=jnp.float32)
    out_ref[...] = (acc + b_ref[...]) * norm_ref[...]


def _linear(h, norm, W, b):
    blk = 1000
    grid = N // blk
    return pl.pallas_call(
        _mm_body,
        grid=(grid,),
        in_specs=[
            pl.BlockSpec((blk, D), lambda i: (i, 0)),
            pl.BlockSpec((blk, 1), lambda i: (i, 0)),
            pl.BlockSpec((D, D), lambda i: (0, 0)),
            pl.BlockSpec((1, D), lambda i: (0, 0)),
        ],
        out_specs=pl.BlockSpec((blk, D), lambda i: (i, 0)),
        out_shape=jax.ShapeDtypeStruct((N, D), jnp.float32),
    )(h, norm, W, b.reshape(1, D))


def _sc_agg_body(hh_hbm, src_hbm, dst_hbm, zeros_hbm, out_hbm,
                 sidx_v, didx_v, rows_v, agg_s, sem):
    c = lax.axis_index("c")
    s = lax.axis_index("s")

    def zero_body(k, carry):
        pltpu.sync_copy(zeros_hbm, agg_s.at[pl.ds(s * ZROWS + k * CHUNK, CHUNK)])
        return carry
    lax.fori_loop(0, ZROWS // CHUNK, zero_body, 0)
    plsc.subcore_barrier()

    w = c * NUM_SUBCORES + s
    base = w * CHUNKS_PER_W * CHUNK

    def edge_body(j, carry):
        off = base + j * CHUNK
        pltpu.sync_copy(src_hbm.at[pl.ds(off, CHUNK)], sidx_v)
        pltpu.sync_copy(dst_hbm.at[pl.ds(off, CHUNK)], didx_v)
        pltpu.async_copy(hh_hbm.at[sidx_v], rows_v, sem).wait()
        pltpu.sync_copy(rows_v, agg_s.at[didx_v], add=True)
        return carry
    lax.fori_loop(0, CHUNKS_PER_W, edge_body, 0)
    plsc.subcore_barrier()

    r0 = s * ZROWS
    pltpu.sync_copy(agg_s.at[pl.ds(r0, ZROWS)],
                    out_hbm.at[c].at[pl.ds(r0, ZROWS)])


_sc_agg = functools.partial(
    pl.kernel,
    out_type=jax.ShapeDtypeStruct((NUM_CORES, AGG_ROWS, D), jnp.float32),
    mesh=plsc.VectorSubcoreMesh(core_axis_name="c", subcore_axis_name="s"),
    scratch_types=[
        pltpu.VMEM((CHUNK,), jnp.int32),
        pltpu.VMEM((CHUNK,), jnp.int32),
        pltpu.VMEM((CHUNK, D), jnp.float32),
        pltpu.VMEM_SHARED((AGG_ROWS, D), jnp.float32),
        pltpu.SemaphoreType.DMA,
    ],
)(_sc_agg_body)


def _comb_body(p_ref, norm_ref, o_ref):
    o_ref[...] = (p_ref[0] + p_ref[1]) * norm_ref[...]


def _combine(partials, norm):
    blk = 1000
    grid = N // blk
    return pl.pallas_call(
        _comb_body,
        grid=(grid,),
        in_specs=[
            pl.BlockSpec((NUM_CORES, blk, D), lambda i: (0, i, 0)),
            pl.BlockSpec((blk, 1), lambda i: (i, 0)),
        ],
        out_specs=pl.BlockSpec((blk, D), lambda i: (i, 0)),
        out_shape=jax.ShapeDtypeStruct((N, D), jnp.float32),
    )(partials, norm)


def kernel(h, norm, W, b, edge_index):
    hh = _linear(h, norm, W, b)

    src = edge_index[0].astype(jnp.int32)
    dst = edge_index[1].astype(jnp.int32)
    pad = E_PAD - E
    src = jnp.concatenate([src, jnp.zeros((pad,), jnp.int32)])
    dst = jnp.concatenate([dst, jnp.full((pad,), DUMMY_DST, jnp.int32)])
    zeros = jnp.zeros((CHUNK, D), jnp.float32)

    partials = _sc_agg(hh, src, dst, zeros)
    return _combine(partials, norm)
